# baseline (speedup 1.0000x reference)
import functools

import jax
import jax.numpy as jnp
from jax import lax
from jax.experimental import pallas as pl
from jax.experimental.pallas import tpu as pltpu
from jax.experimental.pallas import tpu_sc as plsc

_R, _C = 16384, 128
_G = _R * 8
_NC, _NS, _L = 2, 16, 16
_NW = _NC * _NS
_RW = _R // _NW
_GW = _RW * 8
_NB = 4

_mesh = plsc.VectorSubcoreMesh(core_axis_name="c", subcore_axis_name="s")


@functools.partial(
    pl.kernel,
    mesh=_mesh,
    out_type=jax.ShapeDtypeStruct((_G, 16), jnp.float32),
    scratch_types=[
        pltpu.VMEM((_NB, 128), jnp.int32),
        pltpu.VMEM((_NB, 128, 16), jnp.float32),
        pltpu.SemaphoreType.DMA,
        pltpu.SemaphoreType.DMA,
        pltpu.SemaphoreType.DMA,
    ],
    compiler_params=pltpu.CompilerParams(
        needs_layout_passes=False, use_tc_tiling_on_sc=False
    ),
)
def _sc_kernel(x_hbm, out_hbm, idx_v, gbuf, sem_bulk, sem_g, sem_s):
    wid = lax.axis_index("s") * _NC + lax.axis_index("c")
    gbase = wid * _GW

    bulk = pltpu.async_copy(
        x_hbm.at[pl.ds(gbase, _GW)], out_hbm.at[pl.ds(gbase, _GW)], sem_bulk
    )

    lanes = lax.iota(jnp.int32, 16)
    for j in range(_NB):
        for g in range(8):
            idx_v[j, pl.ds(g * _L, _L)] = gbase + ((j * 128 + g * _L) + lanes) * 8 + 7

    gh = [
        pltpu.async_copy(x_hbm.at[idx_v.at[j]], gbuf.at[j], sem_g)
        for j in range(_NB)
    ]

    c12 = jnp.full((16,), 12, jnp.int32)
    c13 = jnp.full((16,), 13, jnp.int32)
    c14 = jnp.full((16,), 14, jnp.int32)
    c15 = jnp.full((16,), 15, jnp.int32)
    for j in range(_NB):
        gh[j].wait()
        cj = jnp.full((16,), j, jnp.int32)
        for g in range(8):
            rows = lanes + g * _L
            b = plsc.load_gather(gbuf, [cj, rows, c12])
            cvb = plsc.load_gather(gbuf, [cj, rows, c13])
            cvl = plsc.load_gather(gbuf, [cj, rows, c14])
            qg = plsc.load_gather(gbuf, [cj, rows, c15])
            c = b / (1.0 / cvb - 1.0)
            d = c / cvl - c
            plsc.store_scatter(gbuf, [cj, rows, c13], c)
            plsc.store_scatter(gbuf, [cj, rows, c14], (1.0 - qg) * d)
            plsc.store_scatter(gbuf, [cj, rows, c15], qg * d)

    bulk.wait()
    sh = [
        pltpu.async_copy(gbuf.at[j], out_hbm.at[idx_v.at[j]], sem_s)
        for j in range(_NB)
    ]
    for h in sh:
        h.wait()


def kernel(x):
    return _sc_kernel(x.reshape(_G, 16)).reshape(_R, _C)


# speedup vs baseline: 10.4647x; 10.4647x over previous
import functools

import jax
import jax.numpy as jnp
from jax import lax
from jax.experimental import pallas as pl
from jax.experimental.pallas import tpu as pltpu
from jax.experimental.pallas import tpu_sc as plsc

_R, _C = 16384, 128
_NC, _NS, _L = 2, 16, 16
_NW = _NC * _NS
_RW = _R // _NW
_NCH = 2
_CH = _RW // _NCH
_CHW = _CH * _C

_mesh = plsc.VectorSubcoreMesh(core_axis_name="c", subcore_axis_name="s")


@functools.partial(
    pl.kernel,
    mesh=_mesh,
    out_type=jax.ShapeDtypeStruct((_R * _C,), jnp.float32),
    scratch_types=[pltpu.VMEM((_NCH * _CHW,), jnp.float32)]
    + [pltpu.SemaphoreType.DMA] * (2 * _NCH),
    compiler_params=pltpu.CompilerParams(needs_layout_passes=False),
)
def _sc_kernel(x_hbm, out_hbm, buf, *sems):
    sin = sems[:_NCH]
    sout = sems[_NCH:]
    wid = lax.axis_index("s") * _NC + lax.axis_index("c")
    base = wid * _RW * _C

    in_h = [
        pltpu.async_copy(
            x_hbm.at[pl.ds(base + i * _CHW, _CHW)],
            buf.at[pl.ds(i * _CHW, _CHW)],
            sin[i],
        )
        for i in range(_NCH)
    ]

    lanes = lax.iota(jnp.int32, 16)

    out_h = []
    for i in range(_NCH):
        in_h[i].wait()

        def _group(g, _):
            row0 = (lanes + (i * _CH + g * _L)) * _C
            b = plsc.load_gather(buf, [row0 + 124])
            cvb = plsc.load_gather(buf, [row0 + 125])
            cvl = plsc.load_gather(buf, [row0 + 126])
            qg = plsc.load_gather(buf, [row0 + 127])
            c = b / (1.0 / cvb - 1.0)
            d = c / cvl - c
            plsc.store_scatter(buf, [row0 + 125], c)
            plsc.store_scatter(buf, [row0 + 126], (1.0 - qg) * d)
            plsc.store_scatter(buf, [row0 + 127], qg * d)
            return 0

        lax.fori_loop(0, _CH // _L, _group, 0)
        out_h.append(
            pltpu.async_copy(
                buf.at[pl.ds(i * _CHW, _CHW)],
                out_hbm.at[pl.ds(base + i * _CHW, _CHW)],
                sout[i],
            )
        )
    for h in out_h:
        h.wait()


def kernel(x):
    return _sc_kernel(x.reshape(_R * _C)).reshape(_R, _C)


# speedup vs baseline: 10.6514x; 1.0178x over previous
import functools

import jax
import jax.numpy as jnp
from jax import lax
from jax.experimental import pallas as pl
from jax.experimental.pallas import tpu as pltpu
from jax.experimental.pallas import tpu_sc as plsc

_R, _C = 16384, 128
_NC, _NS, _L = 2, 16, 16
_NW = _NC * _NS
_RW = _R // _NW
_NCH = 2
_CH = _RW // _NCH

_mesh = plsc.VectorSubcoreMesh(core_axis_name="c", subcore_axis_name="s")


@functools.partial(
    pl.kernel,
    mesh=_mesh,
    out_type=jax.ShapeDtypeStruct((_R, _C), jnp.float32),
    scratch_types=[pltpu.VMEM((_NCH * _CH, _C), jnp.float32)]
    + [pltpu.SemaphoreType.DMA] * (2 * _NCH),
    compiler_params=pltpu.CompilerParams(needs_layout_passes=False),
)
def _sc_kernel(x_hbm, out_hbm, buf, *sems):
    sin = sems[:_NCH]
    sout = sems[_NCH:]
    wid = lax.axis_index("s") * _NC + lax.axis_index("c")
    base = wid * _RW

    in_h = [
        pltpu.async_copy(
            x_hbm.at[pl.ds(base + i * _CH, _CH)],
            buf.at[pl.ds(i * _CH, _CH)],
            sin[i],
        )
        for i in range(_NCH)
    ]

    lanes = lax.iota(jnp.int32, 16)
    c124 = jnp.full((16,), 124, jnp.int32)
    c125 = jnp.full((16,), 125, jnp.int32)
    c126 = jnp.full((16,), 126, jnp.int32)
    c127 = jnp.full((16,), 127, jnp.int32)

    out_h = []
    for i in range(_NCH):
        in_h[i].wait()

        def _group(g, _):
            rows = lanes + (i * _CH + g * _L)
            b = plsc.load_gather(buf, [rows, c124])
            cvb = plsc.load_gather(buf, [rows, c125])
            cvl = plsc.load_gather(buf, [rows, c126])
            qg = plsc.load_gather(buf, [rows, c127])
            c = b / (1.0 / cvb - 1.0)
            d = c / cvl - c
            plsc.store_scatter(buf, [rows, c125], c)
            plsc.store_scatter(buf, [rows, c126], (1.0 - qg) * d)
            plsc.store_scatter(buf, [rows, c127], qg * d)
            return 0

        lax.fori_loop(0, _CH // _L, _group, 0)
        out_h.append(
            pltpu.async_copy(
                buf.at[pl.ds(i * _CH, _CH)],
                out_hbm.at[pl.ds(base + i * _CH, _CH)],
                sout[i],
            )
        )
    for h in out_h:
        h.wait()


def kernel(x):
    return _sc_kernel(x)


# speedup vs baseline: 10.6934x; 1.0039x over previous
import functools

import jax
import jax.numpy as jnp
from jax import lax
from jax.experimental import pallas as pl
from jax.experimental.pallas import tpu as pltpu
from jax.experimental.pallas import tpu_sc as plsc

_R, _C = 16384, 128
_NC, _NS, _L = 2, 16, 16
_NW = _NC * _NS
_RW = _R // _NW
_NCH = 4
_CH = _RW // _NCH

_mesh = plsc.VectorSubcoreMesh(core_axis_name="c", subcore_axis_name="s")


@functools.partial(
    pl.kernel,
    mesh=_mesh,
    out_type=jax.ShapeDtypeStruct((_R, _C), jnp.float32),
    scratch_types=[pltpu.VMEM((_NCH * _CH, _C), jnp.float32)]
    + [pltpu.SemaphoreType.DMA] * (2 * _NCH),
    compiler_params=pltpu.CompilerParams(needs_layout_passes=False),
)
def _sc_kernel(x_hbm, out_hbm, buf, *sems):
    sin = sems[:_NCH]
    sout = sems[_NCH:]
    wid = lax.axis_index("s") * _NC + lax.axis_index("c")
    base = wid * _RW

    in_h = [
        pltpu.async_copy(
            x_hbm.at[pl.ds(base + i * _CH, _CH)],
            buf.at[pl.ds(i * _CH, _CH)],
            sin[i],
        )
        for i in range(_NCH)
    ]

    lanes = lax.iota(jnp.int32, 16)
    c124 = jnp.full((16,), 124, jnp.int32)
    c125 = jnp.full((16,), 125, jnp.int32)
    c126 = jnp.full((16,), 126, jnp.int32)
    c127 = jnp.full((16,), 127, jnp.int32)

    out_h = []
    for i in range(_NCH):
        in_h[i].wait()

        def _group(g, _):
            rows = lanes + (i * _CH + g * _L)
            b = plsc.load_gather(buf, [rows, c124])
            cvb = plsc.load_gather(buf, [rows, c125])
            cvl = plsc.load_gather(buf, [rows, c126])
            qg = plsc.load_gather(buf, [rows, c127])
            c = b / (1.0 / cvb - 1.0)
            d = c / cvl - c
            plsc.store_scatter(buf, [rows, c125], c)
            plsc.store_scatter(buf, [rows, c126], (1.0 - qg) * d)
            plsc.store_scatter(buf, [rows, c127], qg * d)
            return 0

        lax.fori_loop(0, _CH // _L, _group, 0)
        out_h.append(
            pltpu.async_copy(
                buf.at[pl.ds(i * _CH, _CH)],
                out_hbm.at[pl.ds(base + i * _CH, _CH)],
                sout[i],
            )
        )
    for h in out_h:
        h.wait()


def kernel(x):
    return _sc_kernel(x)
